# baseline (speedup 1.0000x reference)
import functools

import jax
import jax.numpy as jnp
from jax import lax
from jax.experimental import pallas as pl
from jax.experimental.pallas import tpu as pltpu
from jax.experimental.pallas import tpu_sc as plsc

_LN2 = 0.6931471805599453
_TN = 400


def _ssp(x):
    m = jnp.maximum(x, 0.0)
    return m + jnp.log(jnp.exp(x - m) + jnp.exp(-m)) - _LN2


def _dot(a, b):
    return jax.lax.dot_general(a, b, (((a.ndim - 1,), (0,)), ((), ())),
                               preferred_element_type=jnp.float32)


def _bdot(a, b):
    return _dot(a.astype(jnp.bfloat16), b.astype(jnp.bfloat16))


def _pack_v(vf):
    h = vf.shape[-1] // 2
    vb = vf.astype(jnp.bfloat16)
    lo = lax.bitcast_convert_type(vb[:, :h], jnp.uint16).astype(jnp.uint32)
    hi = lax.bitcast_convert_type(vb[:, h:], jnp.uint16).astype(jnp.uint32)
    return lo | (hi << 16)


def _unpack_v(vp):
    lo = lax.bitcast_convert_type(vp << 16, jnp.float32)
    hi = lax.bitcast_convert_type(vp & jnp.uint32(0xFFFF0000), jnp.float32)
    return lo, hi


def _sc_gather(table, idx):
    V, Fd = table.shape
    B = idx.shape[0]
    try:
        info = plsc.get_sparse_core_info()
        nc, ns = info.num_cores, info.num_subcores
    except Exception:
        nc, ns = 2, 16
    nw = nc * ns
    per = B // nw
    assert per * nw == B and per % 8 == 0
    ch = 128
    nbuf = 6
    full = per // ch
    groups = full // nbuf
    rest = full - groups * nbuf
    tail = per - full * ch
    mesh = plsc.VectorSubcoreMesh(core_axis_name="c", subcore_axis_name="s")

    @functools.partial(
        pl.kernel, mesh=mesh,
        out_type=jax.ShapeDtypeStruct((B, Fd), table.dtype),
        scratch_types=[
            pltpu.VMEM((per,), jnp.int32),
            pltpu.VMEM((nbuf, ch, Fd), table.dtype),
            pltpu.SemaphoreType.DMA,
            pltpu.SemaphoreType.DMA,
        ],
    )
    def gather(table_hbm, idx_hbm, out_hbm, idx_v, rows_v, sem_g, sem_w):
        wid = lax.axis_index("s") * nc + lax.axis_index("c")
        base = wid * per
        pltpu.sync_copy(idx_hbm.at[pl.ds(base, per)], idx_v)

        def group(g, carry):
            off0 = g * nbuf * ch

            @pl.when(g > 0)
            def _():
                for b in range(nbuf):
                    pltpu.make_async_copy(
                        rows_v.at[b],
                        out_hbm.at[pl.ds(base + off0 + b * ch, ch)],
                        sem_w).wait()

            for b in range(nbuf):
                pltpu.async_copy(
                    table_hbm.at[idx_v.at[pl.ds(off0 + b * ch, ch)]],
                    rows_v.at[b], sem_g)

            for b in range(nbuf):
                pltpu.make_async_copy(
                    table_hbm.at[idx_v.at[pl.ds(off0 + b * ch, ch)]],
                    rows_v.at[b], sem_g).wait()
                pltpu.async_copy(
                    rows_v.at[b],
                    out_hbm.at[pl.ds(base + off0 + b * ch, ch)], sem_w)
            return carry

        lax.fori_loop(0, groups, group, 0)
        for b in range(nbuf):
            pltpu.make_async_copy(
                rows_v.at[b], out_hbm.at[pl.ds(base, ch)], sem_w).wait()
        for r in range(rest):
            off = (groups * nbuf + r) * ch
            pltpu.async_copy(table_hbm.at[idx_v.at[pl.ds(off, ch)]],
                             rows_v.at[0], sem_g).wait()
            pltpu.sync_copy(rows_v.at[0], out_hbm.at[pl.ds(base + off, ch)])
        if tail:
            off = full * ch
            pltpu.async_copy(
                table_hbm.at[idx_v.at[pl.ds(off, tail)]],
                rows_v.at[0].at[pl.ds(0, tail)], sem_g).wait()
            pltpu.sync_copy(rows_v.at[0].at[pl.ds(0, tail)],
                            out_hbm.at[pl.ds(base + off, tail)])

    return gather(table, idx)


def _embed_call(Zi, emb, w_in, b_in):
    N = Zi.shape[0]
    A, Fd = emb.shape
    grid = N // _TN

    def body(z_ref, emb_ref, wi_ref, bi_ref, x_ref, v_ref):
        ar = lax.broadcasted_iota(jnp.int32, (_TN, A), 1)
        onehot = (ar == z_ref[...]).astype(jnp.float32)
        x = _dot(onehot, emb_ref[...])
        x_ref[...] = x
        v_ref[...] = _dot(x, wi_ref[...]) + bi_ref[...]

    return pl.pallas_call(
        body,
        grid=(grid,),
        in_specs=[
            pl.BlockSpec((_TN, 1), lambda i: (i, 0)),
            pl.BlockSpec((A, Fd), lambda i: (0, 0)),
            pl.BlockSpec((Fd, Fd), lambda i: (0, 0)),
            pl.BlockSpec((1, Fd), lambda i: (0, 0)),
        ],
        out_specs=[
            pl.BlockSpec((_TN, Fd), lambda i: (i, 0)),
            pl.BlockSpec((_TN, Fd), lambda i: (i, 0)),
        ],
        out_shape=[
            jax.ShapeDtypeStruct((N, Fd), jnp.float32),
            jax.ShapeDtypeStruct((N, Fd), jnp.float32),
        ],
    )(Zi, emb, w_in, b_in)


def _filter_conv(rbf_ref, vj_ref, x_ref, fw1, fb1, fw2, fb2, w1, b1, w2, b2,
                 K, Fd, R):
    rows = _TN * K
    u = _ssp(_bdot(rbf_ref[...].reshape(rows, R), fw1[...]) + fb1[...])
    w = _bdot(u, fw2[...]) + fb2[...]
    p = w * vj_ref[...]
    y = p.reshape(_TN, K, Fd).sum(axis=1)
    y = _ssp(_bdot(y, w1[...]) + b1[...])
    y = _dot(y, w2[...]) + b2[...]
    return x_ref[...] + y


def _interaction_mid(x, vj, rbf3, blk, nxt, K):
    N, Fd = x.shape
    R = rbf3.shape[-1]
    grid = N // _TN
    rows = _TN * K

    def body(rbf_ref, vj_ref, x_ref, fw1, fb1, fw2, fb2, w1, b1, w2, b2,
             wi, bi, xo_ref, vn_ref):
        xo = _filter_conv(rbf_ref, vj_ref, x_ref, fw1, fb1, fw2, fb2,
                          w1, b1, w2, b2, K, Fd, R)
        xo_ref[...] = xo
        vn_ref[...] = _bdot(xo, wi[...]) + bi[...]

    wspec = lambda s: pl.BlockSpec(s, lambda i: (0, 0))
    return pl.pallas_call(
        body,
        grid=(grid,),
        in_specs=[
            pl.BlockSpec((_TN, K, R), lambda i: (i, 0, 0)),
            pl.BlockSpec((rows, Fd), lambda i: (i, 0)),
            pl.BlockSpec((_TN, Fd), lambda i: (i, 0)),
            wspec((R, Fd)), wspec((1, Fd)), wspec((Fd, Fd)), wspec((1, Fd)),
            wspec((Fd, Fd)), wspec((1, Fd)), wspec((Fd, Fd)), wspec((1, Fd)),
            wspec((Fd, Fd)), wspec((1, Fd)),
        ],
        out_specs=[
            pl.BlockSpec((_TN, Fd), lambda i: (i, 0)),
            pl.BlockSpec((_TN, Fd), lambda i: (i, 0)),
        ],
        out_shape=[
            jax.ShapeDtypeStruct((N, Fd), jnp.float32),
            jax.ShapeDtypeStruct((N, Fd), jnp.float32),
        ],
    )(rbf3, vj, x, blk["fw1"], blk["fb1"].reshape(1, Fd),
      blk["fw2"], blk["fb2"].reshape(1, Fd),
      blk["w1"], blk["b1"].reshape(1, Fd),
      blk["w2"], blk["b2"].reshape(1, Fd),
      nxt["w_in"], nxt["b_in"].reshape(1, Fd))


def _interaction_last(x, vj, rbf3, blk, ro, K):
    N, Fd = x.shape
    R = rbf3.shape[-1]
    H = ro["rw1"].shape[1]
    grid = N // _TN
    rows = _TN * K

    def body(rbf_ref, vj_ref, x_ref, fw1, fb1, fw2, fb2, w1, b1, w2, b2,
             rw1, rb1, rw2, rb2, out_ref, acc_ref):
        xo = _filter_conv(rbf_ref, vj_ref, x_ref, fw1, fb1, fw2, fb2,
                          w1, b1, w2, b2, K, Fd, R)
        h = _ssp(_dot(xo, rw1[...]) + rb1[...])
        i = pl.program_id(0)

        @pl.when(i == 0)
        def _():
            acc_ref[...] = jnp.zeros_like(acc_ref)

        acc_ref[...] += jnp.sum(h, axis=0, keepdims=True)

        @pl.when(i == grid - 1)
        def _():
            out_ref[...] = _dot(acc_ref[...], rw2[...]) + N * rb2[...]

    wspec = lambda s: pl.BlockSpec(s, lambda i: (0, 0))
    return pl.pallas_call(
        body,
        grid=(grid,),
        in_specs=[
            pl.BlockSpec((_TN, K, R), lambda i: (i, 0, 0)),
            pl.BlockSpec((rows, Fd), lambda i: (i, 0)),
            pl.BlockSpec((_TN, Fd), lambda i: (i, 0)),
            wspec((R, Fd)), wspec((1, Fd)), wspec((Fd, Fd)), wspec((1, Fd)),
            wspec((Fd, Fd)), wspec((1, Fd)), wspec((Fd, Fd)), wspec((1, Fd)),
            wspec((Fd, H)), wspec((1, H)), wspec((H, 1)), wspec((1, 1)),
        ],
        out_specs=pl.BlockSpec((1, 1), lambda i: (0, 0)),
        out_shape=jax.ShapeDtypeStruct((1, 1), jnp.float32),
        scratch_shapes=[pltpu.VMEM((1, H), jnp.float32)],
        compiler_params=pltpu.CompilerParams(
            dimension_semantics=("arbitrary",)),
    )(rbf3, vj, x, blk["fw1"], blk["fb1"].reshape(1, Fd),
      blk["fw2"], blk["fb2"].reshape(1, Fd),
      blk["w1"], blk["b1"].reshape(1, Fd),
      blk["w2"], blk["b2"].reshape(1, Fd),
      ro["rw1"], ro["rb1"].reshape(1, H), ro["rw2"],
      ro["rb2"].reshape(1, 1))


def kernel(Z, rbf, neighbors, params):
    emb = params["embedding"]
    blocks = params["blocks"]
    ro = params["readout"]
    N, K = neighbors.shape
    R = rbf.shape[-1]
    Fd = emb.shape[1]
    T = len(blocks)

    nbr = neighbors.reshape(N * K).astype(jnp.int32)
    Zi = Z.astype(jnp.int32).reshape(N, 1)
    rbf_b = rbf.astype(jnp.bfloat16)

    x, v = _embed_call(Zi, emb, blocks[0]["w_in"],
                       blocks[0]["b_in"].reshape(1, Fd))
    for t in range(T - 1):
        vj = _sc_gather(v, nbr)
        x, v = _interaction_mid(x, vj, rbf_b, blocks[t], blocks[t + 1], K)
    vj = _sc_gather(v, nbr)
    e = _interaction_last(x, vj, rbf_b, blocks[T - 1], ro, K)
    return e.reshape(())


# speedup vs baseline: 1.0062x; 1.0062x over previous
import functools

import jax
import jax.numpy as jnp
from jax import lax
from jax.experimental import pallas as pl
from jax.experimental.pallas import tpu as pltpu
from jax.experimental.pallas import tpu_sc as plsc

_LN2 = 0.6931471805599453
_TN = 400


def _ssp(x):
    m = jnp.maximum(x, 0.0)
    return m + jnp.log(jnp.exp(x - m) + jnp.exp(-m)) - _LN2


def _dot(a, b):
    return jax.lax.dot_general(a, b, (((a.ndim - 1,), (0,)), ((), ())),
                               preferred_element_type=jnp.float32)


def _bdot(a, b):
    return _dot(a.astype(jnp.bfloat16), b.astype(jnp.bfloat16))


def _sc_gather(table, idx):
    V, Fd = table.shape
    B = idx.shape[0]
    try:
        info = plsc.get_sparse_core_info()
        nc, ns = info.num_cores, info.num_subcores
    except Exception:
        nc, ns = 2, 16
    nw = nc * ns
    per = B // nw
    assert per * nw == B and per % 8 == 0
    ch = 128
    nbuf = 6
    full = per // ch
    groups = full // nbuf
    rest = full - groups * nbuf
    tail = per - full * ch
    mesh = plsc.VectorSubcoreMesh(core_axis_name="c", subcore_axis_name="s")

    @functools.partial(
        pl.kernel, mesh=mesh,
        out_type=jax.ShapeDtypeStruct((B, Fd), table.dtype),
        scratch_types=[
            pltpu.VMEM((per,), jnp.int32),
            pltpu.VMEM((nbuf, ch, Fd), table.dtype),
            pltpu.SemaphoreType.DMA,
            pltpu.SemaphoreType.DMA,
        ],
    )
    def gather(table_hbm, idx_hbm, out_hbm, idx_v, rows_v, sem_g, sem_w):
        wid = lax.axis_index("s") * nc + lax.axis_index("c")
        base = wid * per
        pltpu.sync_copy(idx_hbm.at[pl.ds(base, per)], idx_v)

        def group(g, carry):
            off0 = g * nbuf * ch

            @pl.when(g > 0)
            def _():
                for b in range(nbuf):
                    pltpu.make_async_copy(
                        rows_v.at[b],
                        out_hbm.at[pl.ds(base + off0 + b * ch, ch)],
                        sem_w).wait()

            for b in range(nbuf):
                pltpu.async_copy(
                    table_hbm.at[idx_v.at[pl.ds(off0 + b * ch, ch)]],
                    rows_v.at[b], sem_g)

            for b in range(nbuf):
                pltpu.make_async_copy(
                    table_hbm.at[idx_v.at[pl.ds(off0 + b * ch, ch)]],
                    rows_v.at[b], sem_g).wait()
                pltpu.async_copy(
                    rows_v.at[b],
                    out_hbm.at[pl.ds(base + off0 + b * ch, ch)], sem_w)
            return carry

        lax.fori_loop(0, groups, group, 0)
        for b in range(nbuf):
            pltpu.make_async_copy(
                rows_v.at[b], out_hbm.at[pl.ds(base, ch)], sem_w).wait()
        for r in range(rest):
            off = (groups * nbuf + r) * ch
            pltpu.async_copy(table_hbm.at[idx_v.at[pl.ds(off, ch)]],
                             rows_v.at[0], sem_g).wait()
            pltpu.sync_copy(rows_v.at[0], out_hbm.at[pl.ds(base + off, ch)])
        if tail:
            off = full * ch
            pltpu.async_copy(
                table_hbm.at[idx_v.at[pl.ds(off, tail)]],
                rows_v.at[0].at[pl.ds(0, tail)], sem_g).wait()
            pltpu.sync_copy(rows_v.at[0].at[pl.ds(0, tail)],
                            out_hbm.at[pl.ds(base + off, tail)])

    return gather(table, idx)


def _embed_call(Zi, emb, w_in, b_in):
    N = Zi.shape[0]
    A, Fd = emb.shape
    grid = N // _TN

    def body(z_ref, emb_ref, wi_ref, bi_ref, x_ref, v_ref):
        ar = lax.broadcasted_iota(jnp.int32, (_TN, A), 1)
        onehot = (ar == z_ref[...]).astype(jnp.float32)
        x = _dot(onehot, emb_ref[...])
        x_ref[...] = x
        v_ref[...] = _dot(x, wi_ref[...]) + bi_ref[...]

    return pl.pallas_call(
        body,
        grid=(grid,),
        in_specs=[
            pl.BlockSpec((_TN, 1), lambda i: (i, 0)),
            pl.BlockSpec((A, Fd), lambda i: (0, 0)),
            pl.BlockSpec((Fd, Fd), lambda i: (0, 0)),
            pl.BlockSpec((1, Fd), lambda i: (0, 0)),
        ],
        out_specs=[
            pl.BlockSpec((_TN, Fd), lambda i: (i, 0)),
            pl.BlockSpec((_TN, Fd), lambda i: (i, 0)),
        ],
        out_shape=[
            jax.ShapeDtypeStruct((N, Fd), jnp.float32),
            jax.ShapeDtypeStruct((N, Fd), jnp.float32),
        ],
    )(Zi, emb, w_in, b_in)


def _filter_conv(rbf_ref, vj_ref, x_ref, fw1, fb1, fw2, fb2, w1, b1, w2, b2,
                 K, Fd, R):
    rows = _TN * K
    u = _ssp(_bdot(rbf_ref[...].reshape(rows, R), fw1[...]) + fb1[...])
    w = _bdot(u, fw2[...]) + fb2[...]
    p = w * vj_ref[...]
    y = p.reshape(_TN, K, Fd).sum(axis=1)
    y = _ssp(_dot(y, w1[...]) + b1[...])
    y = _dot(y, w2[...]) + b2[...]
    return x_ref[...] + y


def _interaction_mid(x, vj, rbf3, blk, nxt, K):
    N, Fd = x.shape
    R = rbf3.shape[-1]
    grid = N // _TN
    rows = _TN * K

    def body(rbf_ref, vj_ref, x_ref, fw1, fb1, fw2, fb2, w1, b1, w2, b2,
             wi, bi, xo_ref, vn_ref):
        xo = _filter_conv(rbf_ref, vj_ref, x_ref, fw1, fb1, fw2, fb2,
                          w1, b1, w2, b2, K, Fd, R)
        xo_ref[...] = xo
        vn_ref[...] = _dot(xo, wi[...]) + bi[...]

    wspec = lambda s: pl.BlockSpec(s, lambda i: (0, 0))
    return pl.pallas_call(
        body,
        grid=(grid,),
        in_specs=[
            pl.BlockSpec((_TN, K, R), lambda i: (i, 0, 0)),
            pl.BlockSpec((rows, Fd), lambda i: (i, 0)),
            pl.BlockSpec((_TN, Fd), lambda i: (i, 0)),
            wspec((R, Fd)), wspec((1, Fd)), wspec((Fd, Fd)), wspec((1, Fd)),
            wspec((Fd, Fd)), wspec((1, Fd)), wspec((Fd, Fd)), wspec((1, Fd)),
            wspec((Fd, Fd)), wspec((1, Fd)),
        ],
        out_specs=[
            pl.BlockSpec((_TN, Fd), lambda i: (i, 0)),
            pl.BlockSpec((_TN, Fd), lambda i: (i, 0)),
        ],
        out_shape=[
            jax.ShapeDtypeStruct((N, Fd), jnp.float32),
            jax.ShapeDtypeStruct((N, Fd), jnp.float32),
        ],
    )(rbf3, vj, x, blk["fw1"], blk["fb1"].reshape(1, Fd),
      blk["fw2"], blk["fb2"].reshape(1, Fd),
      blk["w1"], blk["b1"].reshape(1, Fd),
      blk["w2"], blk["b2"].reshape(1, Fd),
      nxt["w_in"], nxt["b_in"].reshape(1, Fd))


def _interaction_last(x, vj, rbf3, blk, ro, K):
    N, Fd = x.shape
    R = rbf3.shape[-1]
    H = ro["rw1"].shape[1]
    grid = N // _TN
    rows = _TN * K

    def body(rbf_ref, vj_ref, x_ref, fw1, fb1, fw2, fb2, w1, b1, w2, b2,
             rw1, rb1, rw2, rb2, out_ref, acc_ref):
        xo = _filter_conv(rbf_ref, vj_ref, x_ref, fw1, fb1, fw2, fb2,
                          w1, b1, w2, b2, K, Fd, R)
        h = _ssp(_dot(xo, rw1[...]) + rb1[...])
        i = pl.program_id(0)

        @pl.when(i == 0)
        def _():
            acc_ref[...] = jnp.zeros_like(acc_ref)

        acc_ref[...] += jnp.sum(h, axis=0, keepdims=True)

        @pl.when(i == grid - 1)
        def _():
            out_ref[...] = _dot(acc_ref[...], rw2[...]) + N * rb2[...]

    wspec = lambda s: pl.BlockSpec(s, lambda i: (0, 0))
    return pl.pallas_call(
        body,
        grid=(grid,),
        in_specs=[
            pl.BlockSpec((_TN, K, R), lambda i: (i, 0, 0)),
            pl.BlockSpec((rows, Fd), lambda i: (i, 0)),
            pl.BlockSpec((_TN, Fd), lambda i: (i, 0)),
            wspec((R, Fd)), wspec((1, Fd)), wspec((Fd, Fd)), wspec((1, Fd)),
            wspec((Fd, Fd)), wspec((1, Fd)), wspec((Fd, Fd)), wspec((1, Fd)),
            wspec((Fd, H)), wspec((1, H)), wspec((H, 1)), wspec((1, 1)),
        ],
        out_specs=pl.BlockSpec((1, 1), lambda i: (0, 0)),
        out_shape=jax.ShapeDtypeStruct((1, 1), jnp.float32),
        scratch_shapes=[pltpu.VMEM((1, H), jnp.float32)],
        compiler_params=pltpu.CompilerParams(
            dimension_semantics=("arbitrary",)),
    )(rbf3, vj, x, blk["fw1"], blk["fb1"].reshape(1, Fd),
      blk["fw2"], blk["fb2"].reshape(1, Fd),
      blk["w1"], blk["b1"].reshape(1, Fd),
      blk["w2"], blk["b2"].reshape(1, Fd),
      ro["rw1"], ro["rb1"].reshape(1, H), ro["rw2"],
      ro["rb2"].reshape(1, 1))


def kernel(Z, rbf, neighbors, params):
    emb = params["embedding"]
    blocks = params["blocks"]
    ro = params["readout"]
    N, K = neighbors.shape
    R = rbf.shape[-1]
    Fd = emb.shape[1]
    T = len(blocks)

    nbr = neighbors.reshape(N * K).astype(jnp.int32)
    Zi = Z.astype(jnp.int32).reshape(N, 1)
    rbf_b = rbf.astype(jnp.bfloat16)

    x, v = _embed_call(Zi, emb, blocks[0]["w_in"],
                       blocks[0]["b_in"].reshape(1, Fd))
    for t in range(T - 1):
        vj = _sc_gather(v, nbr)
        x, v = _interaction_mid(x, vj, rbf_b, blocks[t], blocks[t + 1], K)
    vj = _sc_gather(v, nbr)
    e = _interaction_last(x, vj, rbf_b, blocks[T - 1], ro, K)
    return e.reshape(())
